# baseline (speedup 1.0000x reference)
import functools

import numpy as np
import jax
import jax.numpy as jnp
from jax import lax
from jax.experimental import pallas as pl
from jax.experimental.pallas import tpu as pltpu
from jax.experimental.pallas import tpu_sc as plsc

N = 10000
E = 320000
D = 128
R = 16

NC = 2
NS = 16
NW = NC * NS
EPW = E // NW
C = 80
NCHUNK = EPW // C
BN = 1000
BP = 2000
NPAD = 10240

_mesh = plsc.VectorSubcoreMesh(
    core_axis_name="c", subcore_axis_name="s", num_cores=NC, num_subcores=NS)

_sc_params = pltpu.CompilerParams(needs_layout_passes=False)
_sc_params_lin = pltpu.CompilerParams(
    needs_layout_passes=False, use_tc_tiling_on_sc=False)

NEG = np.float32(-3e38)


def _fori(n, body, init):
  return lax.fori_loop(np.int32(0), np.int32(n), body, init)


def _tanh(x):
  s = jnp.exp(x + x)
  return 1.0 - 2.0 / (s + 1.0)


@functools.partial(
    pl.kernel,
    out_type=(jax.ShapeDtypeStruct((E,), jnp.float32),
              jax.ShapeDtypeStruct((NW * 16,), jnp.float32),
              jax.ShapeDtypeStruct((NC * NPAD,), jnp.float32)),
    mesh=_mesh,
    compiler_params=_sc_params,
    scratch_types=[
        pltpu.VMEM((EPW,), jnp.int32),
        pltpu.VMEM((EPW,), jnp.int32),
        pltpu.VMEM((EPW,), jnp.int32),
        pltpu.VMEM((NCHUNK, C), jnp.int32),
        pltpu.VMEM((C,), jnp.int32),
        pltpu.VMEM((C,), jnp.int32),
        pltpu.VMEM((C,), jnp.int32),
        pltpu.VMEM((C,), jnp.int32),
        pltpu.VMEM((C, D), jnp.float32),
        pltpu.VMEM((C, D), jnp.float32),
        pltpu.VMEM((C, D), jnp.float32),
        pltpu.VMEM((C, D), jnp.float32),
        pltpu.VMEM((C, D), jnp.float32),
        pltpu.VMEM((C, D), jnp.float32),
        pltpu.VMEM((EPW,), jnp.float32),
        pltpu.VMEM((16,), jnp.float32),
        pltpu.VMEM((NW * 16,), jnp.float32),
        pltpu.VMEM((2048,), jnp.float32),
        pltpu.VMEM_SHARED((NPAD,), jnp.float32),
        pltpu.SemaphoreType.DMA,
        pltpu.SemaphoreType.DMA,
    ],
)
def _sc_att(hr, src, dst, et, efeat, dst3, ex_out, tmax_out, denom_out,
            src_v, dst_v, et_v, dstc_v, it0_v, ih0_v, it1_v, ih1_v,
            t0_v, h0_v, e0_v, t1_v, h1_v, e1_v, att_v, max_v, tmax_v, zbuf,
            shared_d, sem0, sem1):
  c = lax.axis_index("c")
  s = lax.axis_index("s")
  wid = s * NC + c
  base = wid * EPW

  cp1 = pltpu.async_copy(src.at[pl.ds(base, EPW)], src_v, sem0)
  cp2 = pltpu.async_copy(dst.at[pl.ds(base, EPW)], dst_v, sem1)
  cp3 = pltpu.async_copy(et.at[pl.ds(base, EPW)], et_v, sem0)
  cp4 = pltpu.async_copy(dst3.at[wid], dstc_v, sem1)

  def zloop(i, _):
    zbuf[pl.ds(i * 16, 16)] = jnp.zeros((16,), jnp.float32)
    return 0
  _fori(2048 // 16, zloop, 0)

  @pl.when(s < np.int32(NPAD // 2048))
  def _():
    pltpu.sync_copy(zbuf, shared_d.at[pl.ds(s * np.int32(2048), 2048)])

  cp1.wait()
  cp2.wait()
  cp3.wait()
  cp4.wait()

  def fire(k, it_v, ih_v, t_v, h_v, e_v, sem):
    def lane_body(j, _):
      off = k * np.int32(C) + j * np.int32(16)
      e16 = et_v[pl.ds(off, 16)] * np.int32(N)
      it_v[pl.ds(j * 16, 16)] = e16 + src_v[pl.ds(off, 16)]
      ih_v[pl.ds(j * 16, 16)] = e16 + dst_v[pl.ds(off, 16)]
      return 0
    _fori(C // 16, lane_body, 0)
    pltpu.async_copy(hr.at[it_v], t_v, sem)
    pltpu.async_copy(hr.at[ih_v], h_v, sem)
    pltpu.async_copy(efeat.at[pl.ds(base + k * C, C), :], e_v, sem)

  def drain(it_v, ih_v, t_v, h_v, e_v, sem):
    pltpu.make_async_copy(hr.at[it_v], t_v, sem).wait()
    pltpu.make_async_copy(hr.at[ih_v], h_v, sem).wait()
    pltpu.make_async_copy(efeat.at[pl.ds(base, C), :], e_v, sem).wait()

  lanes = lax.iota(jnp.int32, 16)

  def compute(k, t_v, h_v, e_v, m):
    def grp_body(g, m):
      def edge_body(i, carry):
        m, att16 = carry
        for i_off in range(4):
          ei = i * 4 + i_off
          row = g * 16 + ei
          acc = jnp.zeros((16,), jnp.float32)
          for j in range(D // 16):
            sl = pl.ds(j * 16, 16)
            u = h_v[row, sl] + e_v[row, sl]
            w = jnp.exp(u + u)
            acc = acc + t_v[row, sl] * (1.0 - 2.0 / (w + 1.0))
          a = jnp.sum(acc)
          att16 = jnp.where(lanes == ei, a, att16)
          m = jnp.maximum(m, a)
        return m, att16

      m, att16 = lax.fori_loop(0, 4, edge_body,
                               (m, jnp.zeros((16,), jnp.float32)))
      att_v[pl.ds(k * C + g * 16, 16)] = att16
      return m
    return _fori(C // 16, grp_body, m)

  fire(np.int32(0), it0_v, ih0_v, t0_v, h0_v, e0_v, sem0)

  def pair_body(jj, m):
    k0 = jj * np.int32(2)
    fire(k0 + 1, it1_v, ih1_v, t1_v, h1_v, e1_v, sem1)
    drain(it0_v, ih0_v, t0_v, h0_v, e0_v, sem0)
    m = compute(k0, t0_v, h0_v, e0_v, m)
    fire(k0 + 2, it0_v, ih0_v, t0_v, h0_v, e0_v, sem0)
    drain(it1_v, ih1_v, t1_v, h1_v, e1_v, sem1)
    return compute(k0 + 1, t1_v, h1_v, e1_v, m)

  m = _fori(NCHUNK // 2, pair_body, NEG)
  drain(it0_v, ih0_v, t0_v, h0_v, e0_v, sem0)
  m = compute(np.int32(NCHUNK - 1), t0_v, h0_v, e0_v, m)

  max_v[...] = jnp.full((16,), 1.0, jnp.float32) * m
  pltpu.sync_copy(max_v, tmax_out.at[pl.ds(wid * np.int32(16), 16)])

  plsc.subcore_barrier()

  pltpu.sync_copy(tmax_out, tmax_v)

  def mloop(s2, m16):
    off = (s2 * np.int32(NC) + c) * np.int32(16)
    return jnp.maximum(m16, tmax_v[pl.ds(off, 16)])
  m16 = _fori(NS, mloop, jnp.full((16,), NEG, jnp.float32))
  Mc = jnp.max(m16)

  def eloop(i, _):
    sl = pl.ds(i * 16, 16)
    att_v[sl] = jnp.exp(att_v[sl] - Mc)
    return 0
  _fori(EPW // 16, eloop, 0)

  pltpu.sync_copy(att_v, ex_out.at[pl.ds(base, EPW)])

  def scat(k, _):
    pltpu.sync_copy(att_v.at[pl.ds(k * C, C)], shared_d.at[dstc_v.at[k]],
                    add=True)
    return 0
  _fori(NCHUNK, scat, 0)

  plsc.subcore_barrier()

  @pl.when(s < np.int32(NPAD // 2048))
  def _():
    pltpu.sync_copy(shared_d.at[pl.ds(s * np.int32(2048), 2048)], zbuf)
    pltpu.sync_copy(
        zbuf, denom_out.at[pl.ds(c * np.int32(NPAD) + s * np.int32(2048),
                                 2048)])


DH = D // 2


@functools.partial(
    pl.kernel,
    out_type=(jax.ShapeDtypeStruct((NC, N, DH), jnp.float32),
              jax.ShapeDtypeStruct((NC, N, DH), jnp.float32)),
    mesh=_mesh,
    compiler_params=_sc_params_lin,
    scratch_types=[
        pltpu.VMEM((EPW,), jnp.float32),
        pltpu.VMEM((NCHUNK, C), jnp.int32),
        pltpu.VMEM((NCHUNK, C), jnp.int32),
        pltpu.VMEM((C, DH), jnp.float32),
        pltpu.VMEM((C, DH), jnp.float32),
        pltpu.VMEM((C, DH), jnp.float32),
        pltpu.VMEM((C, DH), jnp.float32),
        pltpu.VMEM((N,), jnp.float32),
        pltpu.VMEM((2000,), jnp.float32),
        pltpu.VMEM((NW * 16,), jnp.float32),
        pltpu.VMEM_SHARED((N, DH), jnp.float32),
        pltpu.SemaphoreType.DMA,
        pltpu.SemaphoreType.DMA,
        pltpu.SemaphoreType.DMA,
        pltpu.SemaphoreType.DMA,
    ],
)
def _sc_agg(ex, src3, dst3, denom_p, tmax, nf_lo, nf_hi,
            hn_lo_out, hn_hi_out,
            exall_v, srcc_v, dstc_v, rows0_v, rows1_v, rows2_v, rows3_v,
            d0_v, dtmp_v, tmax_v, shared_h, sem1, sem2, sem3, sem4):
  c = lax.axis_index("c")
  s = lax.axis_index("s")
  wid = s * NC + c
  base = wid * EPW
  nch = N // C

  cp1 = pltpu.async_copy(ex.at[pl.ds(base, EPW)], exall_v, sem1)
  cp2 = pltpu.async_copy(src3.at[wid], srcc_v, sem2)
  cp3 = pltpu.async_copy(dst3.at[wid], dstc_v, sem3)
  cp4 = pltpu.async_copy(denom_p.at[pl.ds(0, N)], d0_v, sem1)
  cp5 = pltpu.async_copy(tmax, tmax_v, sem4)
  cp5.wait()

  def mred(s2, carry):
    m0v, m1v = carry
    off0 = s2 * np.int32(NC * 16)
    m0v = jnp.maximum(m0v, tmax_v[pl.ds(off0, 16)])
    m1v = jnp.maximum(m1v, tmax_v[pl.ds(off0 + np.int32(16), 16)])
    return m0v, m1v
  m0v, m1v = _fori(NS, mred, (jnp.full((16,), NEG, jnp.float32),
                              jnp.full((16,), NEG, jnp.float32)))
  m0 = jnp.max(m0v)
  m1 = jnp.max(m1v)
  M = jnp.maximum(m0, m1)
  ones = jnp.full((16,), 1.0, jnp.float32)
  a0v = jnp.exp(ones * (m0 - M))
  a1v = jnp.exp(ones * (m1 - M))
  mc_own = jnp.where(c == 0, m0, m1)
  aownv = jnp.exp(ones * (mc_own - M))

  cp1.wait()
  cp4.wait()

  def dsum_o(kk, _):
    cpd = pltpu.async_copy(
        denom_p.at[pl.ds(np.int32(NPAD) + kk * np.int32(2000), 2000)],
        dtmp_v, sem2)
    cpd.wait()

    def dsum(i, _):
      sl = pl.ds(i * 16, 16)
      off = kk * np.int32(2000) + i * np.int32(16)
      d0_v[pl.ds(off, 16)] = (d0_v[pl.ds(off, 16)] * a0v
                              + dtmp_v[sl] * a1v)
      return 0
    return _fori(2000 // 16, dsum, 0)
  _fori(N // 2000, dsum_o, 0)

  cp2.wait()
  cp3.wait()

  def adiv_k(k, _):
    def adiv_j(j, _):
      sl = pl.ds(k * C + j * 16, 16)
      dg = plsc.load_gather(d0_v, [dstc_v[k, pl.ds(j * 16, 16)]])
      exall_v[sl] = exall_v[sl] * aownv / dg
      return 0
    return _fori(C // 16, adiv_j, 0)
  _fori(NCHUNK, adiv_k, 0)

  for nf, hn_out in ((nf_lo, hn_lo_out), (nf_hi, hn_hi_out)):
    def zrow(r, _):
      for j in range(DH // 16):
        rows0_v[r, pl.ds(j * 16, 16)] = jnp.zeros((16,), jnp.float32)
      return 0
    _fori(C, zrow, 0)

    def zc(ii, _):
      cid = ii * np.int32(NS) + s

      @pl.when(cid < np.int32(nch))
      def _():
        pltpu.sync_copy(rows0_v, shared_h.at[pl.ds(cid * np.int32(C), C), :])
      return 0
    _fori((nch + NS - 1) // NS, zc, 0)

    plsc.subcore_barrier()

    slots = ((rows0_v, sem1), (rows1_v, sem2), (rows2_v, sem3),
             (rows3_v, sem4))

    def fire(k, rows_v, sem):
      pltpu.async_copy(nf.at[srcc_v.at[k]], rows_v, sem)

    def drain(rows_v, sem):
      pltpu.make_async_copy(nf.at[srcc_v.at[0]], rows_v, sem).wait()

    def process(k, rows_v):
      def scale(i, _):
        for r_off in range(4):
          row = i * 4 + r_off
          sa = plsc.load_gather(
              exall_v, [jnp.full((16,), k * np.int32(C) + row, jnp.int32)])
          for j in range(DH // 16):
            sl = pl.ds(j * 16, 16)
            rows_v[row, sl] = rows_v[row, sl] * sa
        return 0
      _fori(C // 4, scale, 0)
      pltpu.sync_copy(rows_v, shared_h.at[dstc_v.at[k]], add=True)

    for b in range(3):
      fire(np.int32(b), *slots[b])

    def quad_body(jj, _):
      k0 = jj * np.int32(4)
      for b in range(4):
        k = k0 + b

        @pl.when(k + 3 < np.int32(NCHUNK))
        def _():
          fire(k + 3, *slots[(b + 3) % 4])
        drain(*slots[b])
        process(k, slots[b][0])
      return 0
    _fori(NCHUNK // 4, quad_body, 0)

    drain(*slots[0])
    process(np.int32(NCHUNK - 1), slots[0][0])

    plsc.subcore_barrier()

    nco = N // 200

    def outc(ii, _):
      cid = ii * np.int32(NS) + s

      @pl.when(cid < np.int32(nco))
      def _():
        sl = pl.ds(cid * np.int32(200), 200)
        pltpu.sync_copy(shared_h.at[sl, :], hn_out.at[c, sl, :])
      return 0
    _fori((nco + NS - 1) // NS, outc, 0)

    plsc.subcore_barrier()


def _proj_body(h_ref, w_ref, o_ref):
  o_ref[...] = jnp.dot(h_ref[...], w_ref[0],
                       preferred_element_type=jnp.float32)


def _out_body(h_ref, hnl_ref, hnh_ref, w1_ref, w2_ref, o_ref):
  h = h_ref[...]
  hn = jnp.concatenate([hnl_ref[0] + hnl_ref[1],
                        hnh_ref[0] + hnh_ref[1]], axis=-1)
  dn = (((1,), (1,)), ((), ()))
  x1 = lax.dot_general(h + hn, w1_ref[...], dn,
                       preferred_element_type=jnp.float32)
  x2 = lax.dot_general(h * hn, w2_ref[...], dn,
                       preferred_element_type=jnp.float32)
  o_ref[...] = (jnp.where(x1 >= 0, x1, 0.01 * x1)
                + jnp.where(x2 >= 0, x2, 0.01 * x2))


def kernel(nfeat, efeat, relation_weight, res_fc_w, res_fc2_w,
           edge_index, edge_type):
  with jax.enable_x64(False):
    return _kernel32(nfeat, efeat, relation_weight, res_fc_w, res_fc2_w,
                     edge_index, edge_type)


def _kernel32(nfeat, efeat, relation_weight, res_fc_w, res_fc2_w,
              edge_index, edge_type):
  src = edge_index[0].astype(jnp.int32)
  dst = edge_index[1].astype(jnp.int32)
  et = edge_type.astype(jnp.int32)
  src3 = src.reshape(NW, NCHUNK, C)
  dst3 = dst.reshape(NW, NCHUNK, C)

  hr = pl.pallas_call(
      _proj_body,
      grid=(N // BP, R),
      in_specs=[pl.BlockSpec((BP, D), lambda n, r: (n, 0)),
                pl.BlockSpec((1, D, D), lambda n, r: (r, 0, 0))],
      out_specs=pl.BlockSpec((BP, D), lambda n, r: (r * (N // BP) + n, 0)),
      out_shape=jax.ShapeDtypeStruct((R * N, D), jnp.float32),
  )(nfeat, relation_weight)

  ex, tmax, denom_p = _sc_att(hr, src, dst, et, efeat, dst3)
  nf_lo = nfeat[:, :DH]
  nf_hi = nfeat[:, DH:]
  hn_lo, hn_hi = _sc_agg(ex, src3, dst3, denom_p, tmax, nf_lo, nf_hi)

  out = pl.pallas_call(
      _out_body,
      grid=(N // BN,),
      in_specs=[pl.BlockSpec((BN, D), lambda n: (n, 0)),
                pl.BlockSpec((NC, BN, DH), lambda n: (0, n, 0)),
                pl.BlockSpec((NC, BN, DH), lambda n: (0, n, 0)),
                pl.BlockSpec((D, D), lambda n: (0, 0)),
                pl.BlockSpec((D, D), lambda n: (0, 0))],
      out_specs=pl.BlockSpec((BN, D), lambda n: (n, 0)),
      out_shape=jax.ShapeDtypeStruct((N, D), jnp.float32),
  )(nfeat, hn_lo, hn_hi, res_fc_w, res_fc2_w)
  return out


# speedup vs baseline: 1.1118x; 1.1118x over previous
import functools

import numpy as np
import jax
import jax.numpy as jnp
from jax import lax
from jax.experimental import pallas as pl
from jax.experimental.pallas import tpu as pltpu
from jax.experimental.pallas import tpu_sc as plsc

N = 10000
E = 320000
D = 128
R = 16

NC = 2
NS = 16
NW = NC * NS
EPW = E // NW
C = 80
NCHUNK = EPW // C
BN = 1000
BP = 2000
NPAD = 10240

_mesh = plsc.VectorSubcoreMesh(
    core_axis_name="c", subcore_axis_name="s", num_cores=NC, num_subcores=NS)

_sc_params = pltpu.CompilerParams(needs_layout_passes=False)
_sc_params_lin = pltpu.CompilerParams(
    needs_layout_passes=False, use_tc_tiling_on_sc=False)

NEG = np.float32(-3e38)


def _fori(n, body, init):
  return lax.fori_loop(np.int32(0), np.int32(n), body, init)


def _tanh(x):
  s = jnp.exp(x + x)
  return 1.0 - 2.0 / (s + 1.0)


@functools.partial(
    pl.kernel,
    out_type=(jax.ShapeDtypeStruct((E,), jnp.float32),
              jax.ShapeDtypeStruct((NW * 16,), jnp.float32),
              jax.ShapeDtypeStruct((NC * NPAD,), jnp.float32)),
    mesh=_mesh,
    compiler_params=_sc_params,
    scratch_types=[
        pltpu.VMEM((EPW,), jnp.int32),
        pltpu.VMEM((EPW,), jnp.int32),
        pltpu.VMEM((EPW,), jnp.int32),
        pltpu.VMEM((NCHUNK, C), jnp.int32),
        pltpu.VMEM((C,), jnp.int32),
        pltpu.VMEM((C,), jnp.int32),
        pltpu.VMEM((C,), jnp.int32),
        pltpu.VMEM((C,), jnp.int32),
        pltpu.VMEM((C, D), jnp.float32),
        pltpu.VMEM((C, D), jnp.float32),
        pltpu.VMEM((C, D), jnp.float32),
        pltpu.VMEM((C, D), jnp.float32),
        pltpu.VMEM((C, D), jnp.float32),
        pltpu.VMEM((C, D), jnp.float32),
        pltpu.VMEM((EPW,), jnp.float32),
        pltpu.VMEM((16,), jnp.float32),
        pltpu.VMEM((NW * 16,), jnp.float32),
        pltpu.VMEM((2048,), jnp.float32),
        pltpu.VMEM_SHARED((NPAD,), jnp.float32),
        pltpu.SemaphoreType.DMA,
        pltpu.SemaphoreType.DMA,
    ],
)
def _sc_att(hr, src, dst, et, efeat, dst3, ex_out, tmax_out, denom_out,
            src_v, dst_v, et_v, dstc_v, it0_v, ih0_v, it1_v, ih1_v,
            t0_v, h0_v, e0_v, t1_v, h1_v, e1_v, att_v, max_v, tmax_v, zbuf,
            shared_d, sem0, sem1):
  c = lax.axis_index("c")
  s = lax.axis_index("s")
  wid = s * NC + c
  base = wid * EPW

  cp1 = pltpu.async_copy(src.at[pl.ds(base, EPW)], src_v, sem0)
  cp2 = pltpu.async_copy(dst.at[pl.ds(base, EPW)], dst_v, sem1)
  cp3 = pltpu.async_copy(et.at[pl.ds(base, EPW)], et_v, sem0)
  cp4 = pltpu.async_copy(dst3.at[wid], dstc_v, sem1)

  def zloop(i, _):
    zbuf[pl.ds(i * 16, 16)] = jnp.zeros((16,), jnp.float32)
    return 0
  _fori(2048 // 16, zloop, 0)

  @pl.when(s < np.int32(NPAD // 2048))
  def _():
    pltpu.sync_copy(zbuf, shared_d.at[pl.ds(s * np.int32(2048), 2048)])

  cp1.wait()
  cp2.wait()
  cp3.wait()
  cp4.wait()

  def fire(k, it_v, ih_v, t_v, h_v, e_v, sem):
    def lane_body(j, _):
      off = k * np.int32(C) + j * np.int32(16)
      e16 = et_v[pl.ds(off, 16)] * np.int32(N)
      it_v[pl.ds(j * 16, 16)] = e16 + src_v[pl.ds(off, 16)]
      ih_v[pl.ds(j * 16, 16)] = e16 + dst_v[pl.ds(off, 16)]
      return 0
    _fori(C // 16, lane_body, 0)
    pltpu.async_copy(hr.at[it_v], t_v, sem)
    pltpu.async_copy(hr.at[ih_v], h_v, sem)
    pltpu.async_copy(efeat.at[pl.ds(base + k * C, C), :], e_v, sem)

  def drain(it_v, ih_v, t_v, h_v, e_v, sem):
    pltpu.make_async_copy(hr.at[it_v], t_v, sem).wait()
    pltpu.make_async_copy(hr.at[ih_v], h_v, sem).wait()
    pltpu.make_async_copy(efeat.at[pl.ds(base, C), :], e_v, sem).wait()

  lanes = lax.iota(jnp.int32, 16)

  def compute(k, t_v, h_v, e_v, m):
    def grp_body(g, m):
      def edge_body(i, carry):
        m, att16 = carry
        for i_off in range(2):
          ei = i * 2 + i_off
          row = g * 16 + ei
          acc = jnp.zeros((16,), jnp.float32)
          for j in range(D // 16):
            sl = pl.ds(j * 16, 16)
            u = h_v[row, sl] + e_v[row, sl]
            w = jnp.exp(u + u)
            acc = acc + t_v[row, sl] * (1.0 - 2.0 / (w + 1.0))
          a = jnp.sum(acc)
          att16 = jnp.where(lanes == ei, a, att16)
          m = jnp.maximum(m, a)
        return m, att16

      m, att16 = lax.fori_loop(0, 8, edge_body,
                               (m, jnp.zeros((16,), jnp.float32)))
      att_v[pl.ds(k * C + g * 16, 16)] = att16
      return m
    return _fori(C // 16, grp_body, m)

  fire(np.int32(0), it0_v, ih0_v, t0_v, h0_v, e0_v, sem0)

  def pair_body(jj, m):
    k0 = jj * np.int32(2)
    fire(k0 + 1, it1_v, ih1_v, t1_v, h1_v, e1_v, sem1)
    drain(it0_v, ih0_v, t0_v, h0_v, e0_v, sem0)
    m = compute(k0, t0_v, h0_v, e0_v, m)
    fire(k0 + 2, it0_v, ih0_v, t0_v, h0_v, e0_v, sem0)
    drain(it1_v, ih1_v, t1_v, h1_v, e1_v, sem1)
    return compute(k0 + 1, t1_v, h1_v, e1_v, m)

  m = _fori(NCHUNK // 2, pair_body, NEG)
  drain(it0_v, ih0_v, t0_v, h0_v, e0_v, sem0)
  m = compute(np.int32(NCHUNK - 1), t0_v, h0_v, e0_v, m)

  max_v[...] = jnp.full((16,), 1.0, jnp.float32) * m
  pltpu.sync_copy(max_v, tmax_out.at[pl.ds(wid * np.int32(16), 16)])

  plsc.subcore_barrier()

  pltpu.sync_copy(tmax_out, tmax_v)

  def mloop(s2, m16):
    off = (s2 * np.int32(NC) + c) * np.int32(16)
    return jnp.maximum(m16, tmax_v[pl.ds(off, 16)])
  m16 = _fori(NS, mloop, jnp.full((16,), NEG, jnp.float32))
  Mc = jnp.max(m16)

  def eloop(i, _):
    sl = pl.ds(i * 16, 16)
    att_v[sl] = jnp.exp(att_v[sl] - Mc)
    return 0
  _fori(EPW // 16, eloop, 0)

  pltpu.sync_copy(att_v, ex_out.at[pl.ds(base, EPW)])

  def scat(k, _):
    pltpu.sync_copy(att_v.at[pl.ds(k * C, C)], shared_d.at[dstc_v.at[k]],
                    add=True)
    return 0
  _fori(NCHUNK, scat, 0)

  plsc.subcore_barrier()

  @pl.when(s < np.int32(NPAD // 2048))
  def _():
    pltpu.sync_copy(shared_d.at[pl.ds(s * np.int32(2048), 2048)], zbuf)
    pltpu.sync_copy(
        zbuf, denom_out.at[pl.ds(c * np.int32(NPAD) + s * np.int32(2048),
                                 2048)])


DH = D // 2


@functools.partial(
    pl.kernel,
    out_type=(jax.ShapeDtypeStruct((NC, N, DH), jnp.float32),
              jax.ShapeDtypeStruct((NC, N, DH), jnp.float32)),
    mesh=_mesh,
    compiler_params=_sc_params_lin,
    scratch_types=[
        pltpu.VMEM((EPW,), jnp.float32),
        pltpu.VMEM((NCHUNK, C), jnp.int32),
        pltpu.VMEM((NCHUNK, C), jnp.int32),
        pltpu.VMEM((C, DH), jnp.float32),
        pltpu.VMEM((C, DH), jnp.float32),
        pltpu.VMEM((C, DH), jnp.float32),
        pltpu.VMEM((C, DH), jnp.float32),
        pltpu.VMEM((N,), jnp.float32),
        pltpu.VMEM((2000,), jnp.float32),
        pltpu.VMEM((NW * 16,), jnp.float32),
        pltpu.VMEM_SHARED((N, DH), jnp.float32),
        pltpu.SemaphoreType.DMA,
        pltpu.SemaphoreType.DMA,
        pltpu.SemaphoreType.DMA,
        pltpu.SemaphoreType.DMA,
    ],
)
def _sc_agg(ex, src3, dst3, denom_p, tmax, nf_lo, nf_hi,
            hn_lo_out, hn_hi_out,
            exall_v, srcc_v, dstc_v, rows0_v, rows1_v, rows2_v, rows3_v,
            d0_v, dtmp_v, tmax_v, shared_h, sem1, sem2, sem3, sem4):
  c = lax.axis_index("c")
  s = lax.axis_index("s")
  wid = s * NC + c
  base = wid * EPW
  nch = N // C

  cp1 = pltpu.async_copy(ex.at[pl.ds(base, EPW)], exall_v, sem1)
  cp2 = pltpu.async_copy(src3.at[wid], srcc_v, sem2)
  cp3 = pltpu.async_copy(dst3.at[wid], dstc_v, sem3)
  cp4 = pltpu.async_copy(denom_p.at[pl.ds(0, N)], d0_v, sem1)
  cp5 = pltpu.async_copy(tmax, tmax_v, sem4)
  cp5.wait()

  def mred(s2, carry):
    m0v, m1v = carry
    off0 = s2 * np.int32(NC * 16)
    m0v = jnp.maximum(m0v, tmax_v[pl.ds(off0, 16)])
    m1v = jnp.maximum(m1v, tmax_v[pl.ds(off0 + np.int32(16), 16)])
    return m0v, m1v
  m0v, m1v = _fori(NS, mred, (jnp.full((16,), NEG, jnp.float32),
                              jnp.full((16,), NEG, jnp.float32)))
  m0 = jnp.max(m0v)
  m1 = jnp.max(m1v)
  M = jnp.maximum(m0, m1)
  ones = jnp.full((16,), 1.0, jnp.float32)
  a0v = jnp.exp(ones * (m0 - M))
  a1v = jnp.exp(ones * (m1 - M))
  mc_own = jnp.where(c == 0, m0, m1)
  aownv = jnp.exp(ones * (mc_own - M))

  cp1.wait()
  cp4.wait()

  def dsum_o(kk, _):
    cpd = pltpu.async_copy(
        denom_p.at[pl.ds(np.int32(NPAD) + kk * np.int32(2000), 2000)],
        dtmp_v, sem2)
    cpd.wait()

    def dsum(i, _):
      sl = pl.ds(i * 16, 16)
      off = kk * np.int32(2000) + i * np.int32(16)
      d0_v[pl.ds(off, 16)] = (d0_v[pl.ds(off, 16)] * a0v
                              + dtmp_v[sl] * a1v)
      return 0
    return _fori(2000 // 16, dsum, 0)
  _fori(N // 2000, dsum_o, 0)

  cp2.wait()
  cp3.wait()

  def adiv_k(k, _):
    def adiv_j(j, _):
      sl = pl.ds(k * C + j * 16, 16)
      dg = plsc.load_gather(d0_v, [dstc_v[k, pl.ds(j * 16, 16)]])
      exall_v[sl] = exall_v[sl] * aownv / dg
      return 0
    return _fori(C // 16, adiv_j, 0)
  _fori(NCHUNK, adiv_k, 0)

  for nf, hn_out in ((nf_lo, hn_lo_out), (nf_hi, hn_hi_out)):
    def zrow(r, _):
      for j in range(DH // 16):
        rows0_v[r, pl.ds(j * 16, 16)] = jnp.zeros((16,), jnp.float32)
      return 0
    _fori(C, zrow, 0)

    def zc(ii, _):
      cid = ii * np.int32(NS) + s

      @pl.when(cid < np.int32(nch))
      def _():
        pltpu.sync_copy(rows0_v, shared_h.at[pl.ds(cid * np.int32(C), C), :])
      return 0
    _fori((nch + NS - 1) // NS, zc, 0)

    plsc.subcore_barrier()

    slots = ((rows0_v, sem1), (rows1_v, sem2), (rows2_v, sem3),
             (rows3_v, sem4))

    def fire(k, rows_v, sem):
      pltpu.async_copy(nf.at[srcc_v.at[k]], rows_v, sem)

    def drain(rows_v, sem):
      pltpu.make_async_copy(nf.at[srcc_v.at[0]], rows_v, sem).wait()

    def process(k, rows_v):
      def scale(i, _):
        for r_off in range(8):
          row = i * 8 + r_off
          sa = plsc.load_gather(
              exall_v, [jnp.full((16,), k * np.int32(C) + row, jnp.int32)])
          for j in range(DH // 16):
            sl = pl.ds(j * 16, 16)
            rows_v[row, sl] = rows_v[row, sl] * sa
        return 0
      _fori(C // 8, scale, 0)
      pltpu.sync_copy(rows_v, shared_h.at[dstc_v.at[k]], add=True)

    for b in range(3):
      fire(np.int32(b), *slots[b])

    def quad_body(jj, _):
      k0 = jj * np.int32(4)
      for b in range(4):
        k = k0 + b

        @pl.when(k + 3 < np.int32(NCHUNK))
        def _():
          fire(k + 3, *slots[(b + 3) % 4])
        drain(*slots[b])
        process(k, slots[b][0])
      return 0
    _fori(NCHUNK // 4, quad_body, 0)

    drain(*slots[0])
    process(np.int32(NCHUNK - 1), slots[0][0])

    plsc.subcore_barrier()

    nco = N // 200

    def outc(ii, _):
      cid = ii * np.int32(NS) + s

      @pl.when(cid < np.int32(nco))
      def _():
        sl = pl.ds(cid * np.int32(200), 200)
        pltpu.sync_copy(shared_h.at[sl, :], hn_out.at[c, sl, :])
      return 0
    _fori((nco + NS - 1) // NS, outc, 0)

    plsc.subcore_barrier()


def _proj_body(h_ref, w_ref, o_ref):
  o_ref[...] = jnp.dot(h_ref[...], w_ref[0],
                       preferred_element_type=jnp.float32)


def _out_body(h_ref, hnl_ref, hnh_ref, w1_ref, w2_ref, o_ref):
  h = h_ref[...]
  hn = jnp.concatenate([hnl_ref[0] + hnl_ref[1],
                        hnh_ref[0] + hnh_ref[1]], axis=-1)
  dn = (((1,), (1,)), ((), ()))
  x1 = lax.dot_general(h + hn, w1_ref[...], dn,
                       preferred_element_type=jnp.float32)
  x2 = lax.dot_general(h * hn, w2_ref[...], dn,
                       preferred_element_type=jnp.float32)
  o_ref[...] = (jnp.where(x1 >= 0, x1, 0.01 * x1)
                + jnp.where(x2 >= 0, x2, 0.01 * x2))


def kernel(nfeat, efeat, relation_weight, res_fc_w, res_fc2_w,
           edge_index, edge_type):
  with jax.enable_x64(False):
    return _kernel32(nfeat, efeat, relation_weight, res_fc_w, res_fc2_w,
                     edge_index, edge_type)


def _kernel32(nfeat, efeat, relation_weight, res_fc_w, res_fc2_w,
              edge_index, edge_type):
  src = edge_index[0].astype(jnp.int32)
  dst = edge_index[1].astype(jnp.int32)
  et = edge_type.astype(jnp.int32)
  src3 = src.reshape(NW, NCHUNK, C)
  dst3 = dst.reshape(NW, NCHUNK, C)

  hr = pl.pallas_call(
      _proj_body,
      grid=(N // BP, R),
      in_specs=[pl.BlockSpec((BP, D), lambda n, r: (n, 0)),
                pl.BlockSpec((1, D, D), lambda n, r: (r, 0, 0))],
      out_specs=pl.BlockSpec((BP, D), lambda n, r: (r * (N // BP) + n, 0)),
      out_shape=jax.ShapeDtypeStruct((R * N, D), jnp.float32),
  )(nfeat, relation_weight)

  ex, tmax, denom_p = _sc_att(hr, src, dst, et, efeat, dst3)
  nf_lo = nfeat[:, :DH]
  nf_hi = nfeat[:, DH:]
  hn_lo, hn_hi = _sc_agg(ex, src3, dst3, denom_p, tmax, nf_lo, nf_hi)

  out = pl.pallas_call(
      _out_body,
      grid=(N // BN,),
      in_specs=[pl.BlockSpec((BN, D), lambda n: (n, 0)),
                pl.BlockSpec((NC, BN, DH), lambda n: (0, n, 0)),
                pl.BlockSpec((NC, BN, DH), lambda n: (0, n, 0)),
                pl.BlockSpec((D, D), lambda n: (0, 0)),
                pl.BlockSpec((D, D), lambda n: (0, 0))],
      out_specs=pl.BlockSpec((BN, D), lambda n: (n, 0)),
      out_shape=jax.ShapeDtypeStruct((N, D), jnp.float32),
  )(nfeat, hn_lo, hn_hi, res_fc_w, res_fc2_w)
  return out


# speedup vs baseline: 1.1161x; 1.0039x over previous
import functools

import numpy as np
import jax
import jax.numpy as jnp
from jax import lax
from jax.experimental import pallas as pl
from jax.experimental.pallas import tpu as pltpu
from jax.experimental.pallas import tpu_sc as plsc

N = 10000
E = 320000
D = 128
R = 16

NC = 2
NS = 16
NW = NC * NS
EPW = E // NW
C = 80
NCHUNK = EPW // C
BN = 1000
BP = 2000
NPAD = 10240

_mesh = plsc.VectorSubcoreMesh(
    core_axis_name="c", subcore_axis_name="s", num_cores=NC, num_subcores=NS)

_sc_params = pltpu.CompilerParams(needs_layout_passes=False)
_sc_params_lin = pltpu.CompilerParams(
    needs_layout_passes=False, use_tc_tiling_on_sc=False)

NEG = np.float32(-3e38)


def _fori(n, body, init):
  return lax.fori_loop(np.int32(0), np.int32(n), body, init)


def _tanh(x):
  s = jnp.exp(x + x)
  return 1.0 - 2.0 / (s + 1.0)


@functools.partial(
    pl.kernel,
    out_type=(jax.ShapeDtypeStruct((E,), jnp.float32),
              jax.ShapeDtypeStruct((NW * 16,), jnp.float32),
              jax.ShapeDtypeStruct((NC * NPAD,), jnp.float32)),
    mesh=_mesh,
    compiler_params=_sc_params,
    scratch_types=[
        pltpu.VMEM((EPW,), jnp.int32),
        pltpu.VMEM((EPW,), jnp.int32),
        pltpu.VMEM((EPW,), jnp.int32),
        pltpu.VMEM((NCHUNK, C), jnp.int32),
        pltpu.VMEM((C,), jnp.int32),
        pltpu.VMEM((C,), jnp.int32),
        pltpu.VMEM((C,), jnp.int32),
        pltpu.VMEM((C,), jnp.int32),
        pltpu.VMEM((C, D), jnp.float32),
        pltpu.VMEM((C, D), jnp.float32),
        pltpu.VMEM((C, D), jnp.float32),
        pltpu.VMEM((C, D), jnp.float32),
        pltpu.VMEM((C, D), jnp.float32),
        pltpu.VMEM((C, D), jnp.float32),
        pltpu.VMEM((EPW,), jnp.float32),
        pltpu.VMEM((16,), jnp.float32),
        pltpu.VMEM((NW * 16,), jnp.float32),
        pltpu.VMEM((2048,), jnp.float32),
        pltpu.VMEM_SHARED((NPAD,), jnp.float32),
        pltpu.SemaphoreType.DMA,
        pltpu.SemaphoreType.DMA,
    ],
)
def _sc_att(hr, src, dst, et, efeat, dst3, ex_out, tmax_out, denom_out,
            src_v, dst_v, et_v, dstc_v, it0_v, ih0_v, it1_v, ih1_v,
            t0_v, h0_v, e0_v, t1_v, h1_v, e1_v, att_v, max_v, tmax_v, zbuf,
            shared_d, sem0, sem1):
  c = lax.axis_index("c")
  s = lax.axis_index("s")
  wid = s * NC + c
  base = wid * EPW

  cp1 = pltpu.async_copy(src.at[pl.ds(base, EPW)], src_v, sem0)
  cp2 = pltpu.async_copy(dst.at[pl.ds(base, EPW)], dst_v, sem1)
  cp3 = pltpu.async_copy(et.at[pl.ds(base, EPW)], et_v, sem0)
  cp4 = pltpu.async_copy(dst3.at[wid], dstc_v, sem1)

  def zloop(i, _):
    zbuf[pl.ds(i * 16, 16)] = jnp.zeros((16,), jnp.float32)
    return 0
  _fori(2048 // 16, zloop, 0)

  @pl.when(s < np.int32(NPAD // 2048))
  def _():
    pltpu.sync_copy(zbuf, shared_d.at[pl.ds(s * np.int32(2048), 2048)])

  cp1.wait()
  cp2.wait()
  cp3.wait()
  cp4.wait()

  def fire(k, it_v, ih_v, t_v, h_v, e_v, sem):
    def lane_body(j, _):
      off = k * np.int32(C) + j * np.int32(16)
      e16 = et_v[pl.ds(off, 16)] * np.int32(N)
      it_v[pl.ds(j * 16, 16)] = e16 + src_v[pl.ds(off, 16)]
      ih_v[pl.ds(j * 16, 16)] = e16 + dst_v[pl.ds(off, 16)]
      return 0
    _fori(C // 16, lane_body, 0)
    pltpu.async_copy(hr.at[it_v], t_v, sem)
    pltpu.async_copy(hr.at[ih_v], h_v, sem)
    pltpu.async_copy(efeat.at[pl.ds(base + k * C, C), :], e_v, sem)

  def drain(it_v, ih_v, t_v, h_v, e_v, sem):
    pltpu.make_async_copy(hr.at[it_v], t_v, sem).wait()
    pltpu.make_async_copy(hr.at[ih_v], h_v, sem).wait()
    pltpu.make_async_copy(efeat.at[pl.ds(base, C), :], e_v, sem).wait()

  lanes = lax.iota(jnp.int32, 16)

  def compute(k, t_v, h_v, e_v, m):
    def grp_body(g, m):
      def edge_body(i, carry):
        m, att16 = carry
        for i_off in range(2):
          ei = i * 2 + i_off
          row = g * 16 + ei
          acc = jnp.zeros((16,), jnp.float32)
          for j in range(D // 16):
            sl = pl.ds(j * 16, 16)
            u = h_v[row, sl] + e_v[row, sl]
            w = jnp.exp(u + u)
            acc = acc + t_v[row, sl] * (1.0 - 2.0 / (w + 1.0))
          a = jnp.sum(acc)
          att16 = jnp.where(lanes == ei, a, att16)
          m = jnp.maximum(m, a)
        return m, att16

      m, att16 = lax.fori_loop(0, 8, edge_body,
                               (m, jnp.zeros((16,), jnp.float32)))
      att_v[pl.ds(k * C + g * 16, 16)] = att16
      return m
    return _fori(C // 16, grp_body, m)

  fire(np.int32(0), it0_v, ih0_v, t0_v, h0_v, e0_v, sem0)

  def pair_body(jj, m):
    k0 = jj * np.int32(2)
    fire(k0 + 1, it1_v, ih1_v, t1_v, h1_v, e1_v, sem1)
    drain(it0_v, ih0_v, t0_v, h0_v, e0_v, sem0)
    m = compute(k0, t0_v, h0_v, e0_v, m)
    fire(k0 + 2, it0_v, ih0_v, t0_v, h0_v, e0_v, sem0)
    drain(it1_v, ih1_v, t1_v, h1_v, e1_v, sem1)
    return compute(k0 + 1, t1_v, h1_v, e1_v, m)

  m = _fori(NCHUNK // 2, pair_body, NEG)
  drain(it0_v, ih0_v, t0_v, h0_v, e0_v, sem0)
  m = compute(np.int32(NCHUNK - 1), t0_v, h0_v, e0_v, m)

  max_v[...] = jnp.full((16,), 1.0, jnp.float32) * m
  pltpu.sync_copy(max_v, tmax_out.at[pl.ds(wid * np.int32(16), 16)])

  plsc.subcore_barrier()

  pltpu.sync_copy(tmax_out, tmax_v)

  def mloop(s2, m16):
    off = (s2 * np.int32(NC) + c) * np.int32(16)
    return jnp.maximum(m16, tmax_v[pl.ds(off, 16)])
  m16 = _fori(NS, mloop, jnp.full((16,), NEG, jnp.float32))
  Mc = jnp.max(m16)

  def eloop(i, _):
    sl = pl.ds(i * 16, 16)
    att_v[sl] = jnp.exp(att_v[sl] - Mc)
    return 0
  _fori(EPW // 16, eloop, 0)

  pltpu.sync_copy(att_v, ex_out.at[pl.ds(base, EPW)])

  def scat(k, _):
    pltpu.sync_copy(att_v.at[pl.ds(k * C, C)], shared_d.at[dstc_v.at[k]],
                    add=True)
    return 0
  _fori(NCHUNK, scat, 0)

  plsc.subcore_barrier()

  @pl.when(s < np.int32(NPAD // 2048))
  def _():
    pltpu.sync_copy(shared_d.at[pl.ds(s * np.int32(2048), 2048)], zbuf)
    pltpu.sync_copy(
        zbuf, denom_out.at[pl.ds(c * np.int32(NPAD) + s * np.int32(2048),
                                 2048)])


DH = D // 2


@functools.partial(
    pl.kernel,
    out_type=(jax.ShapeDtypeStruct((NC, N, DH), jnp.float32),
              jax.ShapeDtypeStruct((NC, N, DH), jnp.float32)),
    mesh=_mesh,
    compiler_params=_sc_params_lin,
    scratch_types=[
        pltpu.VMEM((EPW,), jnp.float32),
        pltpu.VMEM((NCHUNK, C), jnp.int32),
        pltpu.VMEM((NCHUNK, C), jnp.int32),
        pltpu.VMEM((C, DH), jnp.float32),
        pltpu.VMEM((C, DH), jnp.float32),
        pltpu.VMEM((C, DH), jnp.float32),
        pltpu.VMEM((C, DH), jnp.float32),
        pltpu.VMEM((N,), jnp.float32),
        pltpu.VMEM((2000,), jnp.float32),
        pltpu.VMEM((NW * 16,), jnp.float32),
        pltpu.VMEM_SHARED((N, DH), jnp.float32),
        pltpu.SemaphoreType.DMA,
        pltpu.SemaphoreType.DMA,
        pltpu.SemaphoreType.DMA,
        pltpu.SemaphoreType.DMA,
    ],
)
def _sc_agg(ex, src3, dst3, denom_p, tmax, nf_lo, nf_hi,
            hn_lo_out, hn_hi_out,
            exall_v, srcc_v, dstc_v, rows0_v, rows1_v, rows2_v, rows3_v,
            d0_v, dtmp_v, tmax_v, shared_h, sem1, sem2, sem3, sem4):
  c = lax.axis_index("c")
  s = lax.axis_index("s")
  wid = s * NC + c
  base = wid * EPW
  nch = N // C

  cp1 = pltpu.async_copy(ex.at[pl.ds(base, EPW)], exall_v, sem1)
  cp2 = pltpu.async_copy(src3.at[wid], srcc_v, sem2)
  cp3 = pltpu.async_copy(dst3.at[wid], dstc_v, sem3)
  cp4 = pltpu.async_copy(denom_p.at[pl.ds(0, N)], d0_v, sem1)
  cp5 = pltpu.async_copy(tmax, tmax_v, sem4)
  cp5.wait()

  def mred(s2, carry):
    m0v, m1v = carry
    off0 = s2 * np.int32(NC * 16)
    m0v = jnp.maximum(m0v, tmax_v[pl.ds(off0, 16)])
    m1v = jnp.maximum(m1v, tmax_v[pl.ds(off0 + np.int32(16), 16)])
    return m0v, m1v
  m0v, m1v = _fori(NS, mred, (jnp.full((16,), NEG, jnp.float32),
                              jnp.full((16,), NEG, jnp.float32)))
  m0 = jnp.max(m0v)
  m1 = jnp.max(m1v)
  M = jnp.maximum(m0, m1)
  ones = jnp.full((16,), 1.0, jnp.float32)
  a0v = jnp.exp(ones * (m0 - M))
  a1v = jnp.exp(ones * (m1 - M))
  mc_own = jnp.where(c == 0, m0, m1)
  aownv = jnp.exp(ones * (mc_own - M))

  cp1.wait()
  cp4.wait()

  def dsum_o(kk, _):
    cpd = pltpu.async_copy(
        denom_p.at[pl.ds(np.int32(NPAD) + kk * np.int32(2000), 2000)],
        dtmp_v, sem2)
    cpd.wait()

    def dsum(i, _):
      sl = pl.ds(i * 16, 16)
      off = kk * np.int32(2000) + i * np.int32(16)
      d0_v[pl.ds(off, 16)] = (d0_v[pl.ds(off, 16)] * a0v
                              + dtmp_v[sl] * a1v)
      return 0
    return _fori(2000 // 16, dsum, 0)
  _fori(N // 2000, dsum_o, 0)

  cp2.wait()
  cp3.wait()

  def adiv_k(k, _):
    def adiv_j(j, _):
      sl = pl.ds(k * C + j * 16, 16)
      dg = plsc.load_gather(d0_v, [dstc_v[k, pl.ds(j * 16, 16)]])
      exall_v[sl] = exall_v[sl] * aownv / dg
      return 0
    return _fori(C // 16, adiv_j, 0)
  _fori(NCHUNK, adiv_k, 0)

  for nf, hn_out in ((nf_lo, hn_lo_out), (nf_hi, hn_hi_out)):
    def zrow(r, _):
      for j in range(DH // 16):
        rows0_v[r, pl.ds(j * 16, 16)] = jnp.zeros((16,), jnp.float32)
      return 0
    _fori(C, zrow, 0)

    def zc(ii, _):
      cid = ii * np.int32(NS) + s

      @pl.when(cid < np.int32(nch))
      def _():
        pltpu.sync_copy(rows0_v, shared_h.at[pl.ds(cid * np.int32(C), C), :])
      return 0
    _fori((nch + NS - 1) // NS, zc, 0)

    plsc.subcore_barrier()

    slots = ((rows0_v, sem1), (rows1_v, sem2), (rows2_v, sem3),
             (rows3_v, sem4))

    def fire(k, rows_v, sem):
      pltpu.async_copy(nf.at[srcc_v.at[k]], rows_v, sem)

    def drain(rows_v, sem):
      pltpu.make_async_copy(nf.at[srcc_v.at[0]], rows_v, sem).wait()

    def process(k, rows_v):
      def scale(i, _):
        for r_off in range(4):
          row = i * 4 + r_off
          sa = plsc.load_gather(
              exall_v, [jnp.full((16,), k * np.int32(C) + row, jnp.int32)])
          for j in range(DH // 16):
            sl = pl.ds(j * 16, 16)
            rows_v[row, sl] = rows_v[row, sl] * sa
        return 0
      _fori(C // 4, scale, 0)
      pltpu.sync_copy(rows_v, shared_h.at[dstc_v.at[k]], add=True)

    for b in range(3):
      fire(np.int32(b), *slots[b])

    def quad_body(jj, _):
      k0 = jj * np.int32(4)
      for b in range(4):
        k = k0 + b

        @pl.when(k + 3 < np.int32(NCHUNK))
        def _():
          fire(k + 3, *slots[(b + 3) % 4])
        drain(*slots[b])
        process(k, slots[b][0])
      return 0
    _fori(NCHUNK // 4, quad_body, 0)

    drain(*slots[0])
    process(np.int32(NCHUNK - 1), slots[0][0])

    plsc.subcore_barrier()

    nco = N // 200

    def outc(ii, _):
      cid = ii * np.int32(NS) + s

      @pl.when(cid < np.int32(nco))
      def _():
        sl = pl.ds(cid * np.int32(200), 200)
        pltpu.sync_copy(shared_h.at[sl, :], hn_out.at[c, sl, :])
      return 0
    _fori((nco + NS - 1) // NS, outc, 0)

    plsc.subcore_barrier()


def _proj_body(h_ref, w_ref, o_ref):
  o_ref[...] = jnp.dot(h_ref[...], w_ref[0],
                       preferred_element_type=jnp.float32)


def _out_body(h_ref, hnl_ref, hnh_ref, w1_ref, w2_ref, o_ref):
  h = h_ref[...]
  hn = jnp.concatenate([hnl_ref[0] + hnl_ref[1],
                        hnh_ref[0] + hnh_ref[1]], axis=-1)
  dn = (((1,), (1,)), ((), ()))
  x1 = lax.dot_general(h + hn, w1_ref[...], dn,
                       preferred_element_type=jnp.float32)
  x2 = lax.dot_general(h * hn, w2_ref[...], dn,
                       preferred_element_type=jnp.float32)
  o_ref[...] = (jnp.where(x1 >= 0, x1, 0.01 * x1)
                + jnp.where(x2 >= 0, x2, 0.01 * x2))


def kernel(nfeat, efeat, relation_weight, res_fc_w, res_fc2_w,
           edge_index, edge_type):
  with jax.enable_x64(False):
    return _kernel32(nfeat, efeat, relation_weight, res_fc_w, res_fc2_w,
                     edge_index, edge_type)


def _kernel32(nfeat, efeat, relation_weight, res_fc_w, res_fc2_w,
              edge_index, edge_type):
  src = edge_index[0].astype(jnp.int32)
  dst = edge_index[1].astype(jnp.int32)
  et = edge_type.astype(jnp.int32)
  src3 = src.reshape(NW, NCHUNK, C)
  dst3 = dst.reshape(NW, NCHUNK, C)

  hr = pl.pallas_call(
      _proj_body,
      grid=(N // BP, R),
      in_specs=[pl.BlockSpec((BP, D), lambda n, r: (n, 0)),
                pl.BlockSpec((1, D, D), lambda n, r: (r, 0, 0))],
      out_specs=pl.BlockSpec((BP, D), lambda n, r: (r * (N // BP) + n, 0)),
      out_shape=jax.ShapeDtypeStruct((R * N, D), jnp.float32),
  )(nfeat, relation_weight)

  ex, tmax, denom_p = _sc_att(hr, src, dst, et, efeat, dst3)
  nf_lo = nfeat[:, :DH]
  nf_hi = nfeat[:, DH:]
  hn_lo, hn_hi = _sc_agg(ex, src3, dst3, denom_p, tmax, nf_lo, nf_hi)

  out = pl.pallas_call(
      _out_body,
      grid=(N // BN,),
      in_specs=[pl.BlockSpec((BN, D), lambda n: (n, 0)),
                pl.BlockSpec((NC, BN, DH), lambda n: (0, n, 0)),
                pl.BlockSpec((NC, BN, DH), lambda n: (0, n, 0)),
                pl.BlockSpec((D, D), lambda n: (0, 0)),
                pl.BlockSpec((D, D), lambda n: (0, 0))],
      out_specs=pl.BlockSpec((BN, D), lambda n: (n, 0)),
      out_shape=jax.ShapeDtypeStruct((N, D), jnp.float32),
  )(nfeat, hn_lo, hn_hi, res_fc_w, res_fc2_w)
  return out


# speedup vs baseline: 1.1263x; 1.0091x over previous
import functools

import numpy as np
import jax
import jax.numpy as jnp
from jax import lax
from jax.experimental import pallas as pl
from jax.experimental.pallas import tpu as pltpu
from jax.experimental.pallas import tpu_sc as plsc

N = 10000
E = 320000
D = 128
R = 16

NC = 2
NS = 16
NW = NC * NS
EPW = E // NW
C = 80
NCHUNK = EPW // C
BN = 1000
BP = 2000
NPAD = 10240

_mesh = plsc.VectorSubcoreMesh(
    core_axis_name="c", subcore_axis_name="s", num_cores=NC, num_subcores=NS)

_sc_params = pltpu.CompilerParams(needs_layout_passes=False)
_sc_params_lin = pltpu.CompilerParams(
    needs_layout_passes=False, use_tc_tiling_on_sc=False)

NEG = np.float32(-3e38)


def _fori(n, body, init):
  return lax.fori_loop(np.int32(0), np.int32(n), body, init)


def _tanh(x):
  s = jnp.exp(x + x)
  return 1.0 - 2.0 / (s + 1.0)


@functools.partial(
    pl.kernel,
    out_type=(jax.ShapeDtypeStruct((E,), jnp.float32),
              jax.ShapeDtypeStruct((NW * 16,), jnp.float32),
              jax.ShapeDtypeStruct((NC * NPAD,), jnp.float32)),
    mesh=_mesh,
    compiler_params=_sc_params,
    scratch_types=[
        pltpu.VMEM((EPW,), jnp.int32),
        pltpu.VMEM((EPW,), jnp.int32),
        pltpu.VMEM((EPW,), jnp.int32),
        pltpu.VMEM((NCHUNK, C), jnp.int32),
        pltpu.VMEM((C,), jnp.int32),
        pltpu.VMEM((C,), jnp.int32),
        pltpu.VMEM((C,), jnp.int32),
        pltpu.VMEM((C,), jnp.int32),
        pltpu.VMEM((C, D), jnp.float32),
        pltpu.VMEM((C, D), jnp.float32),
        pltpu.VMEM((C, D), jnp.float32),
        pltpu.VMEM((C, D), jnp.float32),
        pltpu.VMEM((C, D), jnp.float32),
        pltpu.VMEM((C, D), jnp.float32),
        pltpu.VMEM((EPW,), jnp.float32),
        pltpu.VMEM((16,), jnp.float32),
        pltpu.VMEM((NW * 16,), jnp.float32),
        pltpu.VMEM((2048,), jnp.float32),
        pltpu.VMEM_SHARED((NPAD,), jnp.float32),
        pltpu.SemaphoreType.DMA,
        pltpu.SemaphoreType.DMA,
    ],
)
def _sc_att(hr, src, dst, et, efeat, dst3, ex_out, tmax_out, denom_out,
            src_v, dst_v, et_v, dstc_v, it0_v, ih0_v, it1_v, ih1_v,
            t0_v, h0_v, e0_v, t1_v, h1_v, e1_v, att_v, max_v, tmax_v, zbuf,
            shared_d, sem0, sem1):
  c = lax.axis_index("c")
  s = lax.axis_index("s")
  wid = s * NC + c
  base = wid * EPW

  cp1 = pltpu.async_copy(src.at[pl.ds(base, EPW)], src_v, sem0)
  cp2 = pltpu.async_copy(dst.at[pl.ds(base, EPW)], dst_v, sem1)
  cp3 = pltpu.async_copy(et.at[pl.ds(base, EPW)], et_v, sem0)
  cp4 = pltpu.async_copy(dst3.at[wid], dstc_v, sem1)

  def zloop(i, _):
    zbuf[pl.ds(i * 16, 16)] = jnp.zeros((16,), jnp.float32)
    return 0
  _fori(2048 // 16, zloop, 0)

  @pl.when(s < np.int32(NPAD // 2048))
  def _():
    pltpu.sync_copy(zbuf, shared_d.at[pl.ds(s * np.int32(2048), 2048)])

  cp1.wait()
  cp2.wait()
  cp3.wait()
  cp4.wait()

  def fire(k, it_v, ih_v, t_v, h_v, e_v, sem):
    def lane_body(j, _):
      off = k * np.int32(C) + j * np.int32(16)
      e16 = et_v[pl.ds(off, 16)] * np.int32(N)
      it_v[pl.ds(j * 16, 16)] = e16 + src_v[pl.ds(off, 16)]
      ih_v[pl.ds(j * 16, 16)] = e16 + dst_v[pl.ds(off, 16)]
      return 0
    _fori(C // 16, lane_body, 0)
    pltpu.async_copy(hr.at[it_v], t_v, sem)
    pltpu.async_copy(hr.at[ih_v], h_v, sem)
    pltpu.async_copy(efeat.at[pl.ds(base + k * C, C), :], e_v, sem)

  def drain(it_v, ih_v, t_v, h_v, e_v, sem):
    pltpu.make_async_copy(hr.at[it_v], t_v, sem).wait()
    pltpu.make_async_copy(hr.at[ih_v], h_v, sem).wait()
    pltpu.make_async_copy(efeat.at[pl.ds(base, C), :], e_v, sem).wait()

  lanes = lax.iota(jnp.int32, 16)

  def compute(k, t_v, h_v, e_v, m):
    def grp_body(g, m):
      def edge_body(i, carry):
        m, att16 = carry
        for i_off in range(2):
          ei = i * 2 + i_off
          row = g * 16 + ei
          acc = jnp.zeros((16,), jnp.float32)
          for j in range(D // 16):
            sl = pl.ds(j * 16, 16)
            u = h_v[row, sl] + e_v[row, sl]
            w = jnp.exp(u + u)
            acc = acc + t_v[row, sl] * (1.0 - 2.0 / (w + 1.0))
          a = jnp.sum(acc)
          att16 = jnp.where(lanes == ei, a, att16)
          m = jnp.maximum(m, a)
        return m, att16

      m, att16 = lax.fori_loop(0, 8, edge_body,
                               (m, jnp.zeros((16,), jnp.float32)))
      att_v[pl.ds(k * C + g * 16, 16)] = att16
      return m
    return _fori(C // 16, grp_body, m)

  fire(np.int32(0), it0_v, ih0_v, t0_v, h0_v, e0_v, sem0)

  def pair_body(jj, m):
    k0 = jj * np.int32(2)
    fire(k0 + 1, it1_v, ih1_v, t1_v, h1_v, e1_v, sem1)
    drain(it0_v, ih0_v, t0_v, h0_v, e0_v, sem0)
    m = compute(k0, t0_v, h0_v, e0_v, m)
    fire(k0 + 2, it0_v, ih0_v, t0_v, h0_v, e0_v, sem0)
    drain(it1_v, ih1_v, t1_v, h1_v, e1_v, sem1)
    return compute(k0 + 1, t1_v, h1_v, e1_v, m)

  m = _fori(NCHUNK // 2, pair_body, NEG)
  drain(it0_v, ih0_v, t0_v, h0_v, e0_v, sem0)
  m = compute(np.int32(NCHUNK - 1), t0_v, h0_v, e0_v, m)

  max_v[...] = jnp.full((16,), 1.0, jnp.float32) * m
  pltpu.sync_copy(max_v, tmax_out.at[pl.ds(wid * np.int32(16), 16)])

  plsc.subcore_barrier()

  pltpu.sync_copy(tmax_out, tmax_v)

  def mloop(s2, m16):
    off = (s2 * np.int32(NC) + c) * np.int32(16)
    return jnp.maximum(m16, tmax_v[pl.ds(off, 16)])
  m16 = _fori(NS, mloop, jnp.full((16,), NEG, jnp.float32))
  Mc = jnp.max(m16)

  def eloop(i, _):
    for g in range(4):
      sl = pl.ds(i * 64 + g * 16, 16)
      att_v[sl] = jnp.exp(att_v[sl] - Mc)
    return 0
  _fori(EPW // 64, eloop, 0)

  pltpu.sync_copy(att_v, ex_out.at[pl.ds(base, EPW)])

  def scat(k, _):
    pltpu.sync_copy(att_v.at[pl.ds(k * C, C)], shared_d.at[dstc_v.at[k]],
                    add=True)
    return 0
  _fori(NCHUNK, scat, 0)

  plsc.subcore_barrier()

  @pl.when(s < np.int32(NPAD // 2048))
  def _():
    pltpu.sync_copy(shared_d.at[pl.ds(s * np.int32(2048), 2048)], zbuf)
    pltpu.sync_copy(
        zbuf, denom_out.at[pl.ds(c * np.int32(NPAD) + s * np.int32(2048),
                                 2048)])


DH = D // 2


@functools.partial(
    pl.kernel,
    out_type=(jax.ShapeDtypeStruct((NC, N, DH), jnp.float32),
              jax.ShapeDtypeStruct((NC, N, DH), jnp.float32)),
    mesh=_mesh,
    compiler_params=_sc_params_lin,
    scratch_types=[
        pltpu.VMEM((EPW,), jnp.float32),
        pltpu.VMEM((NCHUNK, C), jnp.int32),
        pltpu.VMEM((NCHUNK, C), jnp.int32),
        pltpu.VMEM((C, DH), jnp.float32),
        pltpu.VMEM((C, DH), jnp.float32),
        pltpu.VMEM((C, DH), jnp.float32),
        pltpu.VMEM((C, DH), jnp.float32),
        pltpu.VMEM((N,), jnp.float32),
        pltpu.VMEM((2000,), jnp.float32),
        pltpu.VMEM((NW * 16,), jnp.float32),
        pltpu.VMEM_SHARED((N, DH), jnp.float32),
        pltpu.SemaphoreType.DMA,
        pltpu.SemaphoreType.DMA,
        pltpu.SemaphoreType.DMA,
        pltpu.SemaphoreType.DMA,
    ],
)
def _sc_agg(ex, src3, dst3, denom_p, tmax, nf_lo, nf_hi,
            hn_lo_out, hn_hi_out,
            exall_v, srcc_v, dstc_v, rows0_v, rows1_v, rows2_v, rows3_v,
            d0_v, dtmp_v, tmax_v, shared_h, sem1, sem2, sem3, sem4):
  c = lax.axis_index("c")
  s = lax.axis_index("s")
  wid = s * NC + c
  base = wid * EPW
  nch = N // C

  cp1 = pltpu.async_copy(ex.at[pl.ds(base, EPW)], exall_v, sem1)
  cp2 = pltpu.async_copy(src3.at[wid], srcc_v, sem2)
  cp3 = pltpu.async_copy(dst3.at[wid], dstc_v, sem3)
  cp4 = pltpu.async_copy(denom_p.at[pl.ds(0, N)], d0_v, sem1)
  cp5 = pltpu.async_copy(tmax, tmax_v, sem4)
  cp5.wait()

  def mred(s2, carry):
    m0v, m1v = carry
    off0 = s2 * np.int32(NC * 16)
    m0v = jnp.maximum(m0v, tmax_v[pl.ds(off0, 16)])
    m1v = jnp.maximum(m1v, tmax_v[pl.ds(off0 + np.int32(16), 16)])
    return m0v, m1v
  m0v, m1v = _fori(NS, mred, (jnp.full((16,), NEG, jnp.float32),
                              jnp.full((16,), NEG, jnp.float32)))
  m0 = jnp.max(m0v)
  m1 = jnp.max(m1v)
  M = jnp.maximum(m0, m1)
  ones = jnp.full((16,), 1.0, jnp.float32)
  a0v = jnp.exp(ones * (m0 - M))
  a1v = jnp.exp(ones * (m1 - M))
  mc_own = jnp.where(c == 0, m0, m1)
  aownv = jnp.exp(ones * (mc_own - M))

  cp1.wait()
  cp4.wait()

  def dsum_o(kk, _):
    cpd = pltpu.async_copy(
        denom_p.at[pl.ds(np.int32(NPAD) + kk * np.int32(2000), 2000)],
        dtmp_v, sem2)
    cpd.wait()

    def dsum(i, _):
      sl = pl.ds(i * 16, 16)
      off = kk * np.int32(2000) + i * np.int32(16)
      d0_v[pl.ds(off, 16)] = (d0_v[pl.ds(off, 16)] * a0v
                              + dtmp_v[sl] * a1v)
      return 0
    return _fori(2000 // 16, dsum, 0)
  _fori(N // 2000, dsum_o, 0)

  cp2.wait()
  cp3.wait()

  def adiv_k(k, _):
    for j in range(C // 16):
      sl = pl.ds(k * C + j * 16, 16)
      dg = plsc.load_gather(d0_v, [dstc_v[k, pl.ds(j * 16, 16)]])
      exall_v[sl] = exall_v[sl] * aownv / dg
    return 0
  _fori(NCHUNK, adiv_k, 0)

  for nf, hn_out in ((nf_lo, hn_lo_out), (nf_hi, hn_hi_out)):
    def zrow(r, _):
      for j in range(DH // 16):
        rows0_v[r, pl.ds(j * 16, 16)] = jnp.zeros((16,), jnp.float32)
      return 0
    _fori(C, zrow, 0)

    def zc(ii, _):
      cid = ii * np.int32(NS) + s

      @pl.when(cid < np.int32(nch))
      def _():
        pltpu.sync_copy(rows0_v, shared_h.at[pl.ds(cid * np.int32(C), C), :])
      return 0
    _fori((nch + NS - 1) // NS, zc, 0)

    plsc.subcore_barrier()

    slots = ((rows0_v, sem1), (rows1_v, sem2), (rows2_v, sem3),
             (rows3_v, sem4))

    def fire(k, rows_v, sem):
      pltpu.async_copy(nf.at[srcc_v.at[k]], rows_v, sem)

    def drain(rows_v, sem):
      pltpu.make_async_copy(nf.at[srcc_v.at[0]], rows_v, sem).wait()

    def process(k, rows_v):
      def scale(i, _):
        for r_off in range(4):
          row = i * 4 + r_off
          sa = plsc.load_gather(
              exall_v, [jnp.full((16,), k * np.int32(C) + row, jnp.int32)])
          for j in range(DH // 16):
            sl = pl.ds(j * 16, 16)
            rows_v[row, sl] = rows_v[row, sl] * sa
        return 0
      _fori(C // 4, scale, 0)
      pltpu.sync_copy(rows_v, shared_h.at[dstc_v.at[k]], add=True)

    for b in range(3):
      fire(np.int32(b), *slots[b])

    def quad_body(jj, _):
      k0 = jj * np.int32(4)
      for b in range(4):
        k = k0 + b

        @pl.when(k + 3 < np.int32(NCHUNK))
        def _():
          fire(k + 3, *slots[(b + 3) % 4])
        drain(*slots[b])
        process(k, slots[b][0])
      return 0
    _fori(NCHUNK // 4, quad_body, 0)

    drain(*slots[0])
    process(np.int32(NCHUNK - 1), slots[0][0])

    plsc.subcore_barrier()

    nco = N // 200

    def outc(ii, _):
      cid = ii * np.int32(NS) + s

      @pl.when(cid < np.int32(nco))
      def _():
        sl = pl.ds(cid * np.int32(200), 200)
        pltpu.sync_copy(shared_h.at[sl, :], hn_out.at[c, sl, :])
      return 0
    _fori((nco + NS - 1) // NS, outc, 0)

    plsc.subcore_barrier()


def _proj_body(h_ref, w_ref, o_ref):
  o_ref[...] = jnp.dot(h_ref[...], w_ref[0],
                       preferred_element_type=jnp.float32)


def _out_body(h_ref, hnl_ref, hnh_ref, w1_ref, w2_ref, o_ref):
  h = h_ref[...]
  hn = jnp.concatenate([hnl_ref[0] + hnl_ref[1],
                        hnh_ref[0] + hnh_ref[1]], axis=-1)
  dn = (((1,), (1,)), ((), ()))
  x1 = lax.dot_general(h + hn, w1_ref[...], dn,
                       preferred_element_type=jnp.float32)
  x2 = lax.dot_general(h * hn, w2_ref[...], dn,
                       preferred_element_type=jnp.float32)
  o_ref[...] = (jnp.where(x1 >= 0, x1, 0.01 * x1)
                + jnp.where(x2 >= 0, x2, 0.01 * x2))


def kernel(nfeat, efeat, relation_weight, res_fc_w, res_fc2_w,
           edge_index, edge_type):
  with jax.enable_x64(False):
    return _kernel32(nfeat, efeat, relation_weight, res_fc_w, res_fc2_w,
                     edge_index, edge_type)


def _kernel32(nfeat, efeat, relation_weight, res_fc_w, res_fc2_w,
              edge_index, edge_type):
  src = edge_index[0].astype(jnp.int32)
  dst = edge_index[1].astype(jnp.int32)
  et = edge_type.astype(jnp.int32)
  src3 = src.reshape(NW, NCHUNK, C)
  dst3 = dst.reshape(NW, NCHUNK, C)

  hr = pl.pallas_call(
      _proj_body,
      grid=(N // BP, R),
      in_specs=[pl.BlockSpec((BP, D), lambda n, r: (n, 0)),
                pl.BlockSpec((1, D, D), lambda n, r: (r, 0, 0))],
      out_specs=pl.BlockSpec((BP, D), lambda n, r: (r * (N // BP) + n, 0)),
      out_shape=jax.ShapeDtypeStruct((R * N, D), jnp.float32),
  )(nfeat, relation_weight)

  ex, tmax, denom_p = _sc_att(hr, src, dst, et, efeat, dst3)
  nf_lo = nfeat[:, :DH]
  nf_hi = nfeat[:, DH:]
  hn_lo, hn_hi = _sc_agg(ex, src3, dst3, denom_p, tmax, nf_lo, nf_hi)

  out = pl.pallas_call(
      _out_body,
      grid=(N // BN,),
      in_specs=[pl.BlockSpec((BN, D), lambda n: (n, 0)),
                pl.BlockSpec((NC, BN, DH), lambda n: (0, n, 0)),
                pl.BlockSpec((NC, BN, DH), lambda n: (0, n, 0)),
                pl.BlockSpec((D, D), lambda n: (0, 0)),
                pl.BlockSpec((D, D), lambda n: (0, 0))],
      out_specs=pl.BlockSpec((BN, D), lambda n: (n, 0)),
      out_shape=jax.ShapeDtypeStruct((N, D), jnp.float32),
  )(nfeat, hn_lo, hn_hi, res_fc_w, res_fc2_w)
  return out
